# baseline (device time: 276479 ns/iter reference)
import os

import jax
import jax.numpy as jnp
from jax import lax
from jax.experimental import pallas as pl
from jax.experimental.pallas import tpu as pltpu

N_DEV = 32
M, K, N = 4096, 4096, 2048
CHUNK = M // N_DEV
N_RINGS = 8
CW = N // N_RINGS
HALF = N // 2

PERM = [0, 3, 4, 7, 15, 12, 11, 8, 16, 19, 20, 23, 31, 28, 27, 24,
        25, 26, 29, 30, 22, 21, 18, 17, 9, 10, 13, 14, 6, 5, 2, 1]
assert sorted(PERM) == list(range(32))

N_STEPS = 2 * (N_DEV - 1)

PHASES = int(os.environ.get("KPHASES", "3"))


def kernel(x, w_mat):
    m_per, k_per = x.shape
    _, n = w_mat.shape

    def body(x_ref, w_ref, out_ref, comm, qbuf, amax_buf,
             send_sems, recv_sems, credit_sems, amax_send, amax_recv,
             ag_send, ag_recv, ag_credit):
        my = lax.axis_index("i")

        q = jnp.int32(0)
        nxt = jnp.int32(0)
        prv = jnp.int32(0)
        for j in range(N_DEV):
            hit = my == PERM[j]
            q = jnp.where(hit, j, q)
            nxt = jnp.where(hit, PERM[(j + 1) % N_DEV], nxt)
            prv = jnp.where(hit, PERM[(j - 1) % N_DEV], prv)

        barrier = pltpu.get_barrier_semaphore()
        pl.semaphore_signal(barrier, inc=1, device_id=(prv,),
                            device_id_type=pl.DeviceIdType.MESH)
        pl.semaphore_signal(barrier, inc=1, device_id=(nxt,),
                            device_id_type=pl.DeviceIdType.MESH)
        pl.semaphore_wait(barrier, 2)

        def rows(c):
            return pl.ds(c * CHUNK, CHUNK)

        def mod(v):
            return lax.rem(v + 2 * N_DEV, N_DEV)

        def ring_params(r):
            fwd = r < N_RINGS // 2
            return fwd, r * CW, (nxt if fwd else prv), (prv if fwd else nxt)

        def gemm_piece(c, col0, width):
            out_ref[rows(c), pl.ds(col0, width)] = jnp.dot(
                x_ref[rows(c), :], w_ref[:, pl.ds(col0, width)],
                preferred_element_type=jnp.float32,
            )

        pending = {}

        def credit_wait(r, g):
            if g >= 2:
                pl.semaphore_wait(credit_sems.at[r], 1)

        credit_limit = N_DEV - 4

        def credit_signal(r, g):
            if g <= credit_limit:
                _, _, _, src_dev = ring_params(r)
                pl.semaphore_signal(credit_sems.at[r], inc=1,
                                    device_id=(src_dev,),
                                    device_id_type=pl.DeviceIdType.MESH)

        def drain_send(r, g):
            if g >= 2 and (g - 2) in pending:
                pending[g - 2][r].wait_send()

        def rs_start(r, g):
            fwd, c0, tgt, _ = ring_params(r)
            c_send = mod(q - g) if fwd else mod(q + g)
            rdma = pltpu.make_async_remote_copy(
                src_ref=out_ref.at[rows(c_send), pl.ds(c0, CW)],
                dst_ref=comm.at[r, g % 2],
                send_sem=send_sems.at[r, g % 2],
                recv_sem=recv_sems.at[r, g % 2],
                device_id=(tgt,),
                device_id_type=pl.DeviceIdType.MESH,
            )
            rdma.start()
            pending.setdefault(g, {})[r] = rdma

        gemm_piece(q, 0, N)
        for r in range(N_RINGS):
            rs_start(r, 0)
        for g in range(N_DEV - 1):
            slot = g % 2
            gemm_piece(mod(q - g - 1), 0, HALF)
            gemm_piece(mod(q + g + 1), HALF, HALF)
            for r in range(N_RINGS):
                fwd, c0, _, _ = ring_params(r)
                c_recv = mod(q - g - 1) if fwd else mod(q + g + 1)
                pending[g][r].wait_recv()
                out_ref[rows(c_recv), pl.ds(c0, CW)] = (
                    out_ref[rows(c_recv), pl.ds(c0, CW)]
                    + comm[r, slot, :, :]
                )
                credit_signal(r, g)
                if g < N_DEV - 2:
                    credit_wait(r, g + 1)
                    drain_send(r, g + 1)
                    rs_start(r, g + 1)

        if PHASES < 2:
            for g in (N_DEV - 3, N_DEV - 2):
                for rdma in pending[g].values():
                    rdma.wait_send()
            return

        c_own_f = mod(q + 1)
        c_own_b = mod(q - 1)

        amax = jnp.maximum(
            jnp.max(jnp.abs(out_ref[rows(c_own_f), pl.ds(0, HALF)])),
            jnp.max(jnp.abs(out_ref[rows(c_own_b), pl.ds(HALF, HALF)])),
        )
        amax_buf[pl.ds(my, 1), :] = jnp.full((1, 128), amax, jnp.float32)
        amax_rdmas = []
        for d in range(1, N_DEV):
            peer = mod(my + d)
            rdma = pltpu.make_async_remote_copy(
                src_ref=amax_buf.at[pl.ds(my, 1), :],
                dst_ref=amax_buf.at[pl.ds(my, 1), :],
                send_sem=amax_send.at[0],
                recv_sem=amax_recv.at[0],
                device_id=(peer,),
                device_id_type=pl.DeviceIdType.MESH,
            )
            rdma.start()
            amax_rdmas.append(rdma)
        for rdma in amax_rdmas:
            rdma.wait_send()
        for rdma in amax_rdmas:
            rdma.wait_recv()
        for g in (N_DEV - 3, N_DEV - 2):
            for rdma in pending[g].values():
                rdma.wait_send()
        g_amax = jnp.max(amax_buf[:, :])
        scale = g_amax / 448.0

        for c_own, col0 in ((c_own_f, 0), (c_own_b, HALF)):
            sl = (rows(c_own), pl.ds(col0, HALF))
            qbuf[sl] = (out_ref[sl] / scale).astype(jnp.float8_e4m3fn)
            out_ref[sl] = qbuf[sl].astype(jnp.float32) * scale

        if PHASES < 3:
            return

        LANE_T = (N_DEV // 2, N_DEV // 2, N_DEV // 2 - 1, N_DEV // 2 - 1)

        def lane_slice(lane, c):
            col0 = 0 if lane % 2 == 0 else HALF
            return (rows(c), pl.ds(col0, HALF))

        def lane_send_chunk(lane, t):
            return [mod(q + 1 - t), mod(q - 1 - t),
                    mod(q + 1 + t), mod(q - 1 + t)][lane]

        def lane_recv_chunk(lane, t):
            return [mod(q - t), mod(q - 2 - t),
                    mod(q + 2 + t), mod(q + t)][lane]

        pending_ag = {}

        def ag_start(lane, t):
            sl = lane_slice(lane, lane_send_chunk(lane, t))
            rdma = pltpu.make_async_remote_copy(
                src_ref=qbuf.at[sl],
                dst_ref=qbuf.at[sl],
                send_sem=ag_send.at[lane, t % 2],
                recv_sem=ag_recv.at[lane, t % 2],
                device_id=((nxt if lane < 2 else prv),),
                device_id_type=pl.DeviceIdType.MESH,
            )
            rdma.start()
            pending_ag.setdefault(t, {})[lane] = rdma

        for lane in range(4):
            ag_start(lane, 0)
        for t in range(N_DEV // 2):
            for lane in range(4):
                if t >= LANE_T[lane]:
                    continue
                pending_ag[t][lane].wait_recv()
                if t <= LANE_T[lane] - 3:
                    pl.semaphore_signal(
                        ag_credit.at[lane], inc=1,
                        device_id=((prv if lane < 2 else nxt),),
                        device_id_type=pl.DeviceIdType.MESH)
                if t + 1 < LANE_T[lane]:
                    if t + 1 >= 2:
                        pl.semaphore_wait(ag_credit.at[lane], 1)
                    if t >= 1:
                        pending_ag[t - 1][lane].wait_send()
                    ag_start(lane, t + 1)
                sl = lane_slice(lane, lane_recv_chunk(lane, t))
                out_ref[sl] = qbuf[sl].astype(jnp.float32) * scale
        for lane in range(4):
            for t in (LANE_T[lane] - 2, LANE_T[lane] - 1):
                pending_ag[t][lane].wait_send()

    return pl.pallas_call(
        body,
        out_shape=jax.ShapeDtypeStruct((m_per, n), jnp.float32),
        in_specs=[
            pl.BlockSpec(memory_space=pltpu.VMEM),
            pl.BlockSpec(memory_space=pltpu.VMEM),
        ],
        out_specs=pl.BlockSpec(memory_space=pltpu.VMEM),
        scratch_shapes=[
            pltpu.VMEM((N_RINGS, 2, CHUNK, CW), jnp.float32),
            pltpu.VMEM((m_per, n), jnp.float8_e4m3fn),
            pltpu.VMEM((N_DEV, 128), jnp.float32),
            pltpu.SemaphoreType.DMA((N_RINGS, 2)),
            pltpu.SemaphoreType.DMA((N_RINGS, 2)),
            pltpu.SemaphoreType.REGULAR((N_RINGS,)),
            pltpu.SemaphoreType.DMA((1,)),
            pltpu.SemaphoreType.DMA((1,)),
            pltpu.SemaphoreType.DMA((4, 2)),
            pltpu.SemaphoreType.DMA((4, 2)),
            pltpu.SemaphoreType.REGULAR((4,)),
        ],
        compiler_params=pltpu.CompilerParams(
            collective_id=0,
            vmem_limit_bytes=64 * 1024 * 1024,
        ),
    )(x, w_mat)


# device time: 275766 ns/iter; 1.0026x vs baseline; 1.0026x over previous
import os

import jax
import jax.numpy as jnp
from jax import lax
from jax.experimental import pallas as pl
from jax.experimental.pallas import tpu as pltpu

N_DEV = 32
M, K, N = 4096, 4096, 2048
CHUNK = M // N_DEV
N_RINGS = 4
CW = N // N_RINGS
HALF = N // 2

PERM = [0, 3, 4, 7, 15, 12, 11, 8, 16, 19, 20, 23, 31, 28, 27, 24,
        25, 26, 29, 30, 22, 21, 18, 17, 9, 10, 13, 14, 6, 5, 2, 1]
assert sorted(PERM) == list(range(32))

N_STEPS = 2 * (N_DEV - 1)

PHASES = int(os.environ.get("KPHASES", "3"))


def kernel(x, w_mat):
    m_per, k_per = x.shape
    _, n = w_mat.shape

    def body(x_ref, w_ref, out_ref, comm, qbuf, amax_buf,
             send_sems, recv_sems, credit_sems, amax_send, amax_recv,
             ag_send, ag_recv, ag_credit):
        my = lax.axis_index("i")

        q = jnp.int32(0)
        nxt = jnp.int32(0)
        prv = jnp.int32(0)
        for j in range(N_DEV):
            hit = my == PERM[j]
            q = jnp.where(hit, j, q)
            nxt = jnp.where(hit, PERM[(j + 1) % N_DEV], nxt)
            prv = jnp.where(hit, PERM[(j - 1) % N_DEV], prv)

        barrier = pltpu.get_barrier_semaphore()
        pl.semaphore_signal(barrier, inc=1, device_id=(prv,),
                            device_id_type=pl.DeviceIdType.MESH)
        pl.semaphore_signal(barrier, inc=1, device_id=(nxt,),
                            device_id_type=pl.DeviceIdType.MESH)
        pl.semaphore_wait(barrier, 2)

        def rows(c):
            return pl.ds(c * CHUNK, CHUNK)

        def mod(v):
            return lax.rem(v + 2 * N_DEV, N_DEV)

        def ring_params(r):
            fwd = r < N_RINGS // 2
            return fwd, r * CW, (nxt if fwd else prv), (prv if fwd else nxt)

        def gemm_piece(c, col0, width):
            out_ref[rows(c), pl.ds(col0, width)] = jnp.dot(
                x_ref[rows(c), :], w_ref[:, pl.ds(col0, width)],
                preferred_element_type=jnp.float32,
            )

        pending = {}

        def credit_wait(r, g):
            if g >= 2:
                pl.semaphore_wait(credit_sems.at[r], 1)

        credit_limit = N_DEV - 4

        def credit_signal(r, g):
            if g <= credit_limit:
                _, _, _, src_dev = ring_params(r)
                pl.semaphore_signal(credit_sems.at[r], inc=1,
                                    device_id=(src_dev,),
                                    device_id_type=pl.DeviceIdType.MESH)

        def drain_send(r, g):
            if g >= 2 and (g - 2) in pending:
                pending[g - 2][r].wait_send()

        def rs_start(r, g):
            fwd, c0, tgt, _ = ring_params(r)
            c_send = mod(q - g) if fwd else mod(q + g)
            rdma = pltpu.make_async_remote_copy(
                src_ref=out_ref.at[rows(c_send), pl.ds(c0, CW)],
                dst_ref=comm.at[r, g % 2],
                send_sem=send_sems.at[r, g % 2],
                recv_sem=recv_sems.at[r, g % 2],
                device_id=(tgt,),
                device_id_type=pl.DeviceIdType.MESH,
            )
            rdma.start()
            pending.setdefault(g, {})[r] = rdma

        gemm_piece(q, 0, N)
        for r in range(N_RINGS):
            rs_start(r, 0)
        for g in range(N_DEV - 1):
            slot = g % 2
            gemm_piece(mod(q - g - 1), 0, HALF)
            gemm_piece(mod(q + g + 1), HALF, HALF)
            for r in range(N_RINGS):
                fwd, c0, _, _ = ring_params(r)
                c_recv = mod(q - g - 1) if fwd else mod(q + g + 1)
                pending[g][r].wait_recv()
                out_ref[rows(c_recv), pl.ds(c0, CW)] = (
                    out_ref[rows(c_recv), pl.ds(c0, CW)]
                    + comm[r, slot, :, :]
                )
                credit_signal(r, g)
                if g < N_DEV - 2:
                    credit_wait(r, g + 1)
                    drain_send(r, g + 1)
                    rs_start(r, g + 1)

        if PHASES < 2:
            for g in (N_DEV - 3, N_DEV - 2):
                for rdma in pending[g].values():
                    rdma.wait_send()
            return

        c_own_f = mod(q + 1)
        c_own_b = mod(q - 1)

        amax = jnp.maximum(
            jnp.max(jnp.abs(out_ref[rows(c_own_f), pl.ds(0, HALF)])),
            jnp.max(jnp.abs(out_ref[rows(c_own_b), pl.ds(HALF, HALF)])),
        )
        amax_buf[pl.ds(my, 1), :] = jnp.full((1, 128), amax, jnp.float32)
        amax_rdmas = []
        for d in range(1, N_DEV):
            peer = mod(my + d)
            rdma = pltpu.make_async_remote_copy(
                src_ref=amax_buf.at[pl.ds(my, 1), :],
                dst_ref=amax_buf.at[pl.ds(my, 1), :],
                send_sem=amax_send.at[0],
                recv_sem=amax_recv.at[0],
                device_id=(peer,),
                device_id_type=pl.DeviceIdType.MESH,
            )
            rdma.start()
            amax_rdmas.append(rdma)
        for rdma in amax_rdmas:
            rdma.wait_send()
        for rdma in amax_rdmas:
            rdma.wait_recv()
        for g in (N_DEV - 3, N_DEV - 2):
            for rdma in pending[g].values():
                rdma.wait_send()
        g_amax = jnp.max(amax_buf[:, :])
        scale = g_amax / 448.0

        for c_own, col0 in ((c_own_f, 0), (c_own_b, HALF)):
            sl = (rows(c_own), pl.ds(col0, HALF))
            qbuf[sl] = (out_ref[sl] / scale).astype(jnp.float8_e4m3fn)
            out_ref[sl] = qbuf[sl].astype(jnp.float32) * scale

        if PHASES < 3:
            return

        LANE_T = (N_DEV // 2, N_DEV // 2, N_DEV // 2 - 1, N_DEV // 2 - 1)

        def lane_slice(lane, c):
            col0 = 0 if lane % 2 == 0 else HALF
            return (rows(c), pl.ds(col0, HALF))

        def lane_send_chunk(lane, t):
            return [mod(q + 1 - t), mod(q - 1 - t),
                    mod(q + 1 + t), mod(q - 1 + t)][lane]

        def lane_recv_chunk(lane, t):
            return [mod(q - t), mod(q - 2 - t),
                    mod(q + 2 + t), mod(q + t)][lane]

        pending_ag = {}

        def ag_start(lane, t):
            sl = lane_slice(lane, lane_send_chunk(lane, t))
            rdma = pltpu.make_async_remote_copy(
                src_ref=qbuf.at[sl],
                dst_ref=qbuf.at[sl],
                send_sem=ag_send.at[lane, t % 2],
                recv_sem=ag_recv.at[lane, t % 2],
                device_id=((nxt if lane < 2 else prv),),
                device_id_type=pl.DeviceIdType.MESH,
            )
            rdma.start()
            pending_ag.setdefault(t, {})[lane] = rdma

        for lane in range(4):
            ag_start(lane, 0)
        for t in range(N_DEV // 2):
            for lane in range(4):
                if t >= LANE_T[lane]:
                    continue
                pending_ag[t][lane].wait_recv()
                if t <= LANE_T[lane] - 3:
                    pl.semaphore_signal(
                        ag_credit.at[lane], inc=1,
                        device_id=((prv if lane < 2 else nxt),),
                        device_id_type=pl.DeviceIdType.MESH)
                if t + 1 < LANE_T[lane]:
                    if t + 1 >= 2:
                        pl.semaphore_wait(ag_credit.at[lane], 1)
                    if t >= 1:
                        pending_ag[t - 1][lane].wait_send()
                    ag_start(lane, t + 1)
                sl = lane_slice(lane, lane_recv_chunk(lane, t))
                out_ref[sl] = qbuf[sl].astype(jnp.float32) * scale
        for lane in range(4):
            for t in (LANE_T[lane] - 2, LANE_T[lane] - 1):
                pending_ag[t][lane].wait_send()

    return pl.pallas_call(
        body,
        out_shape=jax.ShapeDtypeStruct((m_per, n), jnp.float32),
        in_specs=[
            pl.BlockSpec(memory_space=pltpu.VMEM),
            pl.BlockSpec(memory_space=pltpu.VMEM),
        ],
        out_specs=pl.BlockSpec(memory_space=pltpu.VMEM),
        scratch_shapes=[
            pltpu.VMEM((N_RINGS, 2, CHUNK, CW), jnp.float32),
            pltpu.VMEM((m_per, n), jnp.float8_e4m3fn),
            pltpu.VMEM((N_DEV, 128), jnp.float32),
            pltpu.SemaphoreType.DMA((N_RINGS, 2)),
            pltpu.SemaphoreType.DMA((N_RINGS, 2)),
            pltpu.SemaphoreType.REGULAR((N_RINGS,)),
            pltpu.SemaphoreType.DMA((1,)),
            pltpu.SemaphoreType.DMA((1,)),
            pltpu.SemaphoreType.DMA((4, 2)),
            pltpu.SemaphoreType.DMA((4, 2)),
            pltpu.SemaphoreType.REGULAR((4,)),
        ],
        compiler_params=pltpu.CompilerParams(
            collective_id=0,
            vmem_limit_bytes=64 * 1024 * 1024,
        ),
    )(x, w_mat)
